# baseline (device time: 128186 ns/iter reference)
import functools

import jax
import jax.numpy as jnp
from jax import lax
from jax.experimental import pallas as pl
from jax.experimental.pallas import tpu as pltpu

N_DEV = 4
B, SQ, SKV_SH, HQ, H_SH, DH = 2, 256, 256, 16, 4, 64
D_MODEL = 512
WINDOW = 128
N_SRC = 2


def kernel(x, Wq, K_ext, V_ext, Wo):
    def body(x_ref, wq_ref, k_ref, v_ref, wo_ref, out_ref,
             kbuf, vbuf, comm, kv_send, kv_recv, ring_send, ring_recv):
        my_pos = lax.axis_index("i")

        barrier_sem = pltpu.get_barrier_semaphore()
        for p in range(1, N_DEV):
            pl.semaphore_signal(
                barrier_sem, inc=1,
                device_id=((my_pos + p) % N_DEV,),
                device_id_type=pl.DeviceIdType.MESH,
            )
        pl.semaphore_wait(barrier_sem, N_DEV - 1)

        for src in range(N_SRC):
            @pl.when(my_pos == src)
            def _(src=src):
                rdmas = []
                targets = [j for j in range(N_DEV) if j != src]
                for ti, j in enumerate(targets):
                    for t, (full, buf) in enumerate(((k_ref, kbuf), (v_ref, vbuf))):
                        rdma = pltpu.make_async_remote_copy(
                            src_ref=full.at[:, :, pl.ds(H_SH * j, H_SH), :],
                            dst_ref=buf.at[src],
                            send_sem=kv_send.at[ti, t],
                            recv_sem=kv_recv.at[src, t],
                            device_id=(j,),
                            device_id_type=pl.DeviceIdType.MESH,
                        )
                        rdma.start()
                        rdmas.append(rdma)
                kbuf[src] = k_ref[:, :, H_SH * src:H_SH * (src + 1), :]
                vbuf[src] = v_ref[:, :, H_SH * src:H_SH * (src + 1), :]
                for r in rdmas:
                    r.wait_send()

            @pl.when(my_pos != src)
            def _(src=src):
                for t, buf in ((0, kbuf), (1, vbuf)):
                    pltpu.make_async_remote_copy(
                        src_ref=k_ref.at[:, :, pl.ds(0, H_SH), :],
                        dst_ref=buf.at[src],
                        send_sem=kv_send.at[0, t],
                        recv_sem=kv_recv.at[src, t],
                        device_id=(src,),
                        device_id_type=pl.DeviceIdType.MESH,
                    ).wait_recv()

        qi = lax.broadcasted_iota(jnp.int32, (SQ, N_SRC * SKV_SH), 0)
        kj = lax.broadcasted_iota(jnp.int32, (SQ, N_SRC * SKV_SH), 1)
        mask = jnp.abs(qi - kj) <= WINDOW

        for b in range(B):
            q_all = jnp.dot(x_ref[b], wq_ref[...],
                            preferred_element_type=jnp.float32)
            ctx_cols = []
            for h in range(H_SH):
                q_h = q_all[:, DH * h:DH * (h + 1)] * 0.125
                k_cat = jnp.concatenate(
                    [kbuf[0, b, :, h, :], kbuf[1, b, :, h, :]], axis=0)
                v_cat = jnp.concatenate(
                    [vbuf[0, b, :, h, :], vbuf[1, b, :, h, :]], axis=0)
                s = lax.dot_general(
                    q_h, k_cat, (((1,), (1,)), ((), ())),
                    preferred_element_type=jnp.float32)
                s = jnp.where(mask, s, -1e9)
                m = jnp.max(s, axis=1, keepdims=True)
                w = jnp.exp(s - m)
                w = w / jnp.sum(w, axis=1, keepdims=True)
                ctx_cols.append(jnp.dot(w, v_cat,
                                        preferred_element_type=jnp.float32))
            ctx_b = jnp.concatenate(ctx_cols, axis=1)
            part_b = jnp.dot(ctx_b, wo_ref[...],
                             preferred_element_type=jnp.float32)
            out_ref[b] = part_b
            comm[0, b] = part_b

        right = (my_pos + 1) % N_DEV
        for hop in range(N_DEV - 1):
            rdma = pltpu.make_async_remote_copy(
                src_ref=comm.at[hop],
                dst_ref=comm.at[hop + 1],
                send_sem=ring_send.at[hop],
                recv_sem=ring_recv.at[hop],
                device_id=(right,),
                device_id_type=pl.DeviceIdType.MESH,
            )
            rdma.start()
            rdma.wait()
            out_ref[...] = out_ref[...] + comm[hop + 1]

        def _exit(second_barrier):
            for p in range(1, N_DEV):
                pl.semaphore_signal(
                    second_barrier, inc=1,
                    device_id=((my_pos + p) % N_DEV,),
                    device_id_type=pl.DeviceIdType.MESH,
                )
            pl.semaphore_wait(second_barrier, N_DEV - 1)
        pl.run_scoped(_exit, second_barrier=pltpu.SemaphoreType.REGULAR)

    return pl.pallas_call(
        body,
        out_shape=jax.ShapeDtypeStruct((B, SQ, D_MODEL), jnp.float32),
        in_specs=[pl.BlockSpec(memory_space=pltpu.VMEM)] * 5,
        out_specs=pl.BlockSpec(memory_space=pltpu.VMEM),
        scratch_shapes=[
            pltpu.VMEM((N_SRC, B, SKV_SH, H_SH, DH), jnp.float32),
            pltpu.VMEM((N_SRC, B, SKV_SH, H_SH, DH), jnp.float32),
            pltpu.VMEM((N_DEV, B, SQ, D_MODEL), jnp.float32),
            pltpu.SemaphoreType.DMA((N_DEV - 1, 2)),
            pltpu.SemaphoreType.DMA((N_SRC, 2)),
            pltpu.SemaphoreType.DMA((N_DEV - 1,)),
            pltpu.SemaphoreType.DMA((N_DEV - 1,)),
        ],
        compiler_params=pltpu.CompilerParams(collective_id=0),
    )(x, Wq, K_ext, V_ext, Wo)


# device time: 40886 ns/iter; 3.1352x vs baseline; 3.1352x over previous
import jax
import jax.numpy as jnp
from jax import lax
from jax.experimental import pallas as pl
from jax.experimental.pallas import tpu as pltpu

N_DEV = 4
B, SQ, SKV_SH, HQ, H_SH, DH = 2, 256, 256, 16, 4, 64
D_MODEL = 512
WINDOW = 128
HD = H_SH * DH
KV1 = 128
SKV = SKV_SH + KV1
QTR = SQ // N_DEV


def kernel(x, Wq, K_ext, V_ext, Wo):
    def body(x_ref, wq_ref, k_ref, v_ref, wo_ref, out_ref,
             kstage, vstage, kbuf0, vbuf0, kbuf1, vbuf1,
             rs_stage, rs_recv, ag_stage, ag_recv,
             kv_send, kv_recv, rs_send_sem, rs_recv_sem,
             ag_send_sem, ag_recv_sem):
        my_pos = lax.axis_index("i")

        barrier_sem = pltpu.get_barrier_semaphore()
        for p in range(1, N_DEV):
            pl.semaphore_signal(
                barrier_sem, inc=1,
                device_id=((my_pos + p) % N_DEV,),
                device_id_type=pl.DeviceIdType.MESH,
            )
        pl.semaphore_wait(barrier_sem, N_DEV - 1)

        def kv_descriptors(src):
            rows = SKV_SH if src == 0 else KV1
            ds = []
            targets = [j for j in range(N_DEV) if j != src]
            for ti, j in enumerate(targets):
                for t, (stage, buf0, buf1) in enumerate(
                        ((kstage, kbuf0, kbuf1), (vstage, vbuf0, vbuf1))):
                    dst = buf0 if src == 0 else buf1
                    ds.append(pltpu.make_async_remote_copy(
                        src_ref=stage.at[:, pl.ds(0, rows), pl.ds(HD * j, HD)],
                        dst_ref=dst,
                        send_sem=kv_send.at[ti, t],
                        recv_sem=kv_recv.at[src, t],
                        device_id=(j,),
                        device_id_type=pl.DeviceIdType.MESH,
                    ))
            return ds

        for src in range(2):
            @pl.when(my_pos == src)
            def _(src=src):
                rows = SKV_SH if src == 0 else KV1
                kstage[:, :rows, :] = k_ref[:, :rows, :, :].astype(
                    jnp.bfloat16).reshape(B, rows, HQ * DH)
                vstage[:, :rows, :] = v_ref[:, :rows, :, :].astype(
                    jnp.bfloat16).reshape(B, rows, HQ * DH)
                for d in kv_descriptors(src):
                    d.start()
                dst_k = kbuf0 if src == 0 else kbuf1
                dst_v = vbuf0 if src == 0 else vbuf1
                dst_k[...] = kstage[:, :rows, HD * src:HD * (src + 1)]
                dst_v[...] = vstage[:, :rows, HD * src:HD * (src + 1)]

        wq_b = wq_ref[...].astype(jnp.bfloat16)
        wo_b = wo_ref[...].astype(jnp.bfloat16)
        q_all = []
        for b in range(B):
            xb = x_ref[b].astype(jnp.bfloat16)
            q_all.append(jnp.dot(xb, wq_b,
                                 preferred_element_type=jnp.float32))

        for src in range(2):
            @pl.when(my_pos != src)
            def _(src=src):
                rows = SKV_SH if src == 0 else KV1
                for t, (buf0, buf1) in enumerate(
                        ((kbuf0, kbuf1), (vbuf0, vbuf1))):
                    dst = buf0 if src == 0 else buf1
                    pltpu.make_async_remote_copy(
                        src_ref=kstage.at[:, pl.ds(0, rows), pl.ds(0, HD)],
                        dst_ref=dst,
                        send_sem=kv_send.at[0, t],
                        recv_sem=kv_recv.at[src, t],
                        device_id=(src,),
                        device_id_type=pl.DeviceIdType.MESH,
                    ).wait_recv()

        qi = lax.broadcasted_iota(jnp.int32, (SQ, SKV), 0)
        kj = lax.broadcasted_iota(jnp.int32, (SQ, SKV), 1)
        mask = jnp.abs(qi - kj) <= WINDOW

        parts = []
        for b in range(B):
            k_cat = jnp.concatenate([kbuf0[b], kbuf1[b]], axis=0)
            v_cat = jnp.concatenate([vbuf0[b], vbuf1[b]], axis=0)
            ctx_cols = []
            for h in range(H_SH):
                q_h = (q_all[b][:, DH * h:DH * (h + 1)] * 0.125).astype(
                    jnp.bfloat16)
                k_h = k_cat[:, DH * h:DH * (h + 1)]
                s = lax.dot_general(
                    q_h, k_h, (((1,), (1,)), ((), ())),
                    preferred_element_type=jnp.float32)
                s = jnp.where(mask, s, -1e9)
                m = jnp.max(s, axis=1, keepdims=True)
                w = jnp.exp(s - m)
                w = (w / jnp.sum(w, axis=1, keepdims=True)).astype(
                    jnp.bfloat16)
                ctx_cols.append(jnp.dot(
                    w, v_cat[:, DH * h:DH * (h + 1)],
                    preferred_element_type=jnp.float32))
            ctx_b = jnp.concatenate(ctx_cols, axis=1).astype(jnp.bfloat16)
            part_b = jnp.dot(ctx_b, wo_b,
                             preferred_element_type=jnp.float32)
            parts.append(part_b)
            rs_stage[b] = part_b.astype(jnp.bfloat16)
            out_ref[b] = part_b

        rs_rdmas = []
        for p in range(1, N_DEV):
            tgt = (my_pos + p) % N_DEV
            rs_rdmas.append(pltpu.make_async_remote_copy(
                src_ref=rs_stage.at[:, pl.ds(tgt * QTR, QTR), :],
                dst_ref=rs_recv.at[p - 1],
                send_sem=rs_send_sem.at[p - 1],
                recv_sem=rs_recv_sem.at[p - 1],
                device_id=(tgt,),
                device_id_type=pl.DeviceIdType.MESH,
            ))
        for r in rs_rdmas:
            r.start()

        for src in range(2):
            @pl.when(my_pos == src)
            def _(src=src):
                for d in kv_descriptors(src):
                    d.wait_send()

        for r in rs_rdmas:
            r.wait_recv()

        for b in range(B):
            red_b = out_ref[b, pl.ds(my_pos * QTR, QTR), :]
            for p in range(1, N_DEV):
                red_b = red_b + rs_recv[p - 1, b].astype(jnp.float32)
            out_ref[b, pl.ds(my_pos * QTR, QTR), :] = red_b
            ag_stage[b] = red_b.astype(jnp.bfloat16)

        ag_rdmas = []
        for p in range(1, N_DEV):
            tgt = (my_pos + p) % N_DEV
            ag_rdmas.append(pltpu.make_async_remote_copy(
                src_ref=ag_stage,
                dst_ref=ag_recv.at[p - 1],
                send_sem=ag_send_sem.at[p - 1],
                recv_sem=ag_recv_sem.at[p - 1],
                device_id=(tgt,),
                device_id_type=pl.DeviceIdType.MESH,
            ))
        for r in ag_rdmas:
            r.start()
        for r in ag_rdmas:
            r.wait_recv()
        for p in range(1, N_DEV):
            src = (my_pos - p) % N_DEV
            for b in range(B):
                out_ref[b, pl.ds(src * QTR, QTR), :] = (
                    ag_recv[p - 1, b].astype(jnp.float32))

        for r in rs_rdmas:
            r.wait_send()
        for r in ag_rdmas:
            r.wait_send()

        def _exit(second_barrier):
            for p in range(1, N_DEV):
                pl.semaphore_signal(
                    second_barrier, inc=1,
                    device_id=((my_pos + p) % N_DEV,),
                    device_id_type=pl.DeviceIdType.MESH,
                )
            pl.semaphore_wait(second_barrier, N_DEV - 1)
        pl.run_scoped(_exit, second_barrier=pltpu.SemaphoreType.REGULAR)

    return pl.pallas_call(
        body,
        out_shape=jax.ShapeDtypeStruct((B, SQ, D_MODEL), jnp.float32),
        in_specs=[pl.BlockSpec(memory_space=pltpu.VMEM)] * 5,
        out_specs=pl.BlockSpec(memory_space=pltpu.VMEM),
        scratch_shapes=[
            pltpu.VMEM((B, SKV_SH, HQ * DH), jnp.bfloat16),
            pltpu.VMEM((B, SKV_SH, HQ * DH), jnp.bfloat16),
            pltpu.VMEM((B, SKV_SH, HD), jnp.bfloat16),
            pltpu.VMEM((B, SKV_SH, HD), jnp.bfloat16),
            pltpu.VMEM((B, KV1, HD), jnp.bfloat16),
            pltpu.VMEM((B, KV1, HD), jnp.bfloat16),
            pltpu.VMEM((B, SQ, D_MODEL), jnp.bfloat16),
            pltpu.VMEM((N_DEV - 1, B, QTR, D_MODEL), jnp.bfloat16),
            pltpu.VMEM((B, QTR, D_MODEL), jnp.bfloat16),
            pltpu.VMEM((N_DEV - 1, B, QTR, D_MODEL), jnp.bfloat16),
            pltpu.SemaphoreType.DMA((N_DEV - 1, 2)),
            pltpu.SemaphoreType.DMA((2, 2)),
            pltpu.SemaphoreType.DMA((N_DEV - 1,)),
            pltpu.SemaphoreType.DMA((N_DEV - 1,)),
            pltpu.SemaphoreType.DMA((N_DEV - 1,)),
            pltpu.SemaphoreType.DMA((N_DEV - 1,)),
        ],
        compiler_params=pltpu.CompilerParams(collective_id=0),
    )(x, Wq, K_ext, V_ext, Wo)


# device time: 12592 ns/iter; 10.1800x vs baseline; 3.2470x over previous
import contextlib
import os

import jax
import jax.numpy as jnp
from jax import lax
from jax.experimental import pallas as pl
from jax.experimental.pallas import tpu as pltpu

_PROF = os.environ.get("KERNEL_PROF_SCOPES", "0") == "1"
_ABLATE = os.environ.get("KERNEL_ABLATE", "")
_RDMA = _ABLATE != "compute"


def _scope(name):
    return jax.named_scope(name) if _PROF else contextlib.nullcontext()


N_DEV = 4
B, SQ, SKV_SH, HQ, H_SH, DH = 2, 256, 256, 16, 4, 64
D_MODEL = 512
WINDOW = 128
HD = H_SH * DH
KV1 = 128
SKV = SKV_SH + KV1
QTR = SQ // N_DEV


def kernel(x, Wq, K_ext, V_ext, Wo):
    def body(x_ref, wq_ref, k_ref, v_ref, wo_ref, out_ref,
             kstage, vstage, kbuf0, vbuf0, kbuf1, vbuf1,
             rs_stage, rs_recv, ag_stage, ag_recv,
             kv_send, kv_recv, rs_send_sem, rs_recv_sem,
             ag_send_sem, ag_recv_sem):
        my_pos = lax.axis_index("i")

        if not _RDMA:
            kbuf0[...] = k_ref[:, :, 0:4, :].astype(jnp.bfloat16).reshape(
                B, SKV_SH, HD)
            vbuf0[...] = v_ref[:, :, 0:4, :].astype(jnp.bfloat16).reshape(
                B, SKV_SH, HD)
            kbuf1[...] = k_ref[:, :KV1, 0:4, :].astype(jnp.bfloat16).reshape(
                B, KV1, HD)
            vbuf1[...] = v_ref[:, :KV1, 0:4, :].astype(jnp.bfloat16).reshape(
                B, KV1, HD)

        def kv_descriptors(src):
            rows = SKV_SH if src == 0 else KV1
            ds = []
            targets = [j for j in range(N_DEV) if j != src]
            for ti, j in enumerate(targets):
                for t, (stage, buf0, buf1) in enumerate(
                        ((kstage, kbuf0, kbuf1), (vstage, vbuf0, vbuf1))):
                    dst = buf0 if src == 0 else buf1
                    ds.append(pltpu.make_async_remote_copy(
                        src_ref=stage.at[:, pl.ds(0, rows), pl.ds(HD * j, HD)],
                        dst_ref=dst,
                        send_sem=kv_send.at[ti, t],
                        recv_sem=kv_recv.at[src, t],
                        device_id=(j,),
                        device_id_type=pl.DeviceIdType.MESH,
                    ))
            return ds

        if _RDMA:
            with _scope("barrier"):
                barrier_sem = pltpu.get_barrier_semaphore()
                for p in range(1, N_DEV):
                    pl.semaphore_signal(
                        barrier_sem, inc=1,
                        device_id=((my_pos + p) % N_DEV,),
                        device_id_type=pl.DeviceIdType.MESH,
                    )
                pl.semaphore_wait(barrier_sem, N_DEV - 1)

            with _scope("kv_stage_send"):
                for src in range(2):
                    @pl.when(my_pos == src)
                    def _(src=src):
                        rows = SKV_SH if src == 0 else KV1
                        kstage[:, :rows, :] = k_ref[:, :rows, :, :].astype(
                            jnp.bfloat16).reshape(B, rows, HQ * DH)
                        vstage[:, :rows, :] = v_ref[:, :rows, :, :].astype(
                            jnp.bfloat16).reshape(B, rows, HQ * DH)
                        for d in kv_descriptors(src):
                            d.start()
                        dst_k = kbuf0 if src == 0 else kbuf1
                        dst_v = vbuf0 if src == 0 else vbuf1
                        dst_k[...] = kstage[:, :rows, HD * src:HD * (src + 1)]
                        dst_v[...] = vstage[:, :rows, HD * src:HD * (src + 1)]

        with _scope("qproj"):
            wq_b = wq_ref[...].astype(jnp.bfloat16)
            wo_b = wo_ref[...].astype(jnp.bfloat16)
            q_all = []
            for b in range(B):
                xb = x_ref[b].astype(jnp.bfloat16)
                q_all.append(jnp.dot(xb, wq_b,
                                     preferred_element_type=jnp.float32))

        if _RDMA:
            with _scope("kv_wait_recv"):
                for src in range(2):
                    @pl.when(my_pos != src)
                    def _(src=src):
                        rows = SKV_SH if src == 0 else KV1
                        for t, (buf0, buf1) in enumerate(
                                ((kbuf0, kbuf1), (vbuf0, vbuf1))):
                            dst = buf0 if src == 0 else buf1
                            pltpu.make_async_remote_copy(
                                src_ref=kstage.at[:, pl.ds(0, rows),
                                                  pl.ds(0, HD)],
                                dst_ref=dst,
                                send_sem=kv_send.at[0, t],
                                recv_sem=kv_recv.at[src, t],
                                device_id=(src,),
                                device_id_type=pl.DeviceIdType.MESH,
                            ).wait_recv()

        attn_scope = _scope("attn")
        attn_scope.__enter__()
        qi = lax.broadcasted_iota(jnp.int32, (SQ, SKV), 0)
        kj = lax.broadcasted_iota(jnp.int32, (SQ, SKV), 1)
        mask = jnp.abs(qi - kj) <= WINDOW

        parts = []
        for b in range(B):
            k_cat = jnp.concatenate([kbuf0[b], kbuf1[b]], axis=0)
            v_cat = jnp.concatenate([vbuf0[b], vbuf1[b]], axis=0)
            ctx_cols = []
            for h in range(H_SH):
                q_h = (q_all[b][:, DH * h:DH * (h + 1)] * 0.125).astype(
                    jnp.bfloat16)
                k_h = k_cat[:, DH * h:DH * (h + 1)]
                s = lax.dot_general(
                    q_h, k_h, (((1,), (1,)), ((), ())),
                    preferred_element_type=jnp.float32)
                s = jnp.where(mask, s, -1e9)
                m = jnp.max(s, axis=1, keepdims=True)
                w = jnp.exp(s - m)
                w = (w / jnp.sum(w, axis=1, keepdims=True)).astype(
                    jnp.bfloat16)
                ctx_cols.append(jnp.dot(
                    w, v_cat[:, DH * h:DH * (h + 1)],
                    preferred_element_type=jnp.float32))
            ctx_b = jnp.concatenate(ctx_cols, axis=1).astype(jnp.bfloat16)
            part_b = jnp.dot(ctx_b, wo_b,
                             preferred_element_type=jnp.float32)
            parts.append(part_b)
            rs_stage[b] = part_b.astype(jnp.bfloat16)
            out_ref[b] = part_b
        attn_scope.__exit__(None, None, None)

        if _RDMA:
            with _scope("rs_start"):
                rs_rdmas = []
                for p in range(1, N_DEV):
                    tgt = (my_pos + p) % N_DEV
                    rs_rdmas.append(pltpu.make_async_remote_copy(
                        src_ref=rs_stage.at[:, pl.ds(tgt * QTR, QTR), :],
                        dst_ref=rs_recv.at[p - 1],
                        send_sem=rs_send_sem.at[p - 1],
                        recv_sem=rs_recv_sem.at[p - 1],
                        device_id=(tgt,),
                        device_id_type=pl.DeviceIdType.MESH,
                    ))
                for r in rs_rdmas:
                    r.start()

            with _scope("kv_wait_send"):
                for src in range(2):
                    @pl.when(my_pos == src)
                    def _(src=src):
                        for d in kv_descriptors(src):
                            d.wait_send()

            with _scope("rs_wait_recv"):
                for r in rs_rdmas:
                    r.wait_recv()

            with _scope("reduce"):
                for b in range(B):
                    red_b = out_ref[b, pl.ds(my_pos * QTR, QTR), :]
                    for p in range(1, N_DEV):
                        red_b = red_b + rs_recv[p - 1, b].astype(jnp.float32)
                    out_ref[b, pl.ds(my_pos * QTR, QTR), :] = red_b
                    ag_stage[b] = red_b.astype(jnp.bfloat16)

            with _scope("ag_start"):
                ag_rdmas = []
                for p in range(1, N_DEV):
                    tgt = (my_pos + p) % N_DEV
                    ag_rdmas.append(pltpu.make_async_remote_copy(
                        src_ref=ag_stage,
                        dst_ref=ag_recv.at[p - 1],
                        send_sem=ag_send_sem.at[p - 1],
                        recv_sem=ag_recv_sem.at[p - 1],
                        device_id=(tgt,),
                        device_id_type=pl.DeviceIdType.MESH,
                    ))
                for r in ag_rdmas:
                    r.start()
            with _scope("ag_wait_recv"):
                for r in ag_rdmas:
                    r.wait_recv()
            with _scope("assemble"):
                for p in range(1, N_DEV):
                    src = (my_pos - p) % N_DEV
                    for b in range(B):
                        out_ref[b, pl.ds(src * QTR, QTR), :] = (
                            ag_recv[p - 1, b].astype(jnp.float32))

            with _scope("tail_wait_send"):
                for r in rs_rdmas:
                    r.wait_send()
                for r in ag_rdmas:
                    r.wait_send()

            def _exit(second_barrier):
                for p in range(1, N_DEV):
                    pl.semaphore_signal(
                        second_barrier, inc=1,
                        device_id=((my_pos + p) % N_DEV,),
                        device_id_type=pl.DeviceIdType.MESH,
                    )
                pl.semaphore_wait(second_barrier, N_DEV - 1)
            with _scope("exit_barrier"):
                pl.run_scoped(_exit,
                              second_barrier=pltpu.SemaphoreType.REGULAR)

    return pl.pallas_call(
        body,
        out_shape=jax.ShapeDtypeStruct((B, SQ, D_MODEL), jnp.float32),
        in_specs=[pl.BlockSpec(memory_space=pltpu.VMEM)] * 5,
        out_specs=pl.BlockSpec(memory_space=pltpu.VMEM),
        scratch_shapes=[
            pltpu.VMEM((B, SKV_SH, HQ * DH), jnp.bfloat16),
            pltpu.VMEM((B, SKV_SH, HQ * DH), jnp.bfloat16),
            pltpu.VMEM((B, SKV_SH, HD), jnp.bfloat16),
            pltpu.VMEM((B, SKV_SH, HD), jnp.bfloat16),
            pltpu.VMEM((B, KV1, HD), jnp.bfloat16),
            pltpu.VMEM((B, KV1, HD), jnp.bfloat16),
            pltpu.VMEM((B, SQ, D_MODEL), jnp.bfloat16),
            pltpu.VMEM((N_DEV - 1, B, QTR, D_MODEL), jnp.bfloat16),
            pltpu.VMEM((B, QTR, D_MODEL), jnp.bfloat16),
            pltpu.VMEM((N_DEV - 1, B, QTR, D_MODEL), jnp.bfloat16),
            pltpu.SemaphoreType.DMA((N_DEV - 1, 2)),
            pltpu.SemaphoreType.DMA((2, 2)),
            pltpu.SemaphoreType.DMA((N_DEV - 1,)),
            pltpu.SemaphoreType.DMA((N_DEV - 1,)),
            pltpu.SemaphoreType.DMA((N_DEV - 1,)),
            pltpu.SemaphoreType.DMA((N_DEV - 1,)),
        ],
        compiler_params=(pltpu.CompilerParams(collective_id=0) if _RDMA
                         else pltpu.CompilerParams()),
    )(x, Wq, K_ext, V_ext, Wo)
